# Initial kernel scaffold; baseline (speedup 1.0000x reference)
import jax
import jax.numpy as jnp
from jax.experimental import pallas as pl


def kernel(x, edge_index, W_in, b_in, W1, b1, W2, b2, W3, b3, Wl, bl):
    raise NotImplementedError("write your pallas kernel here")


# speedup vs baseline: 12.2880x; 12.2880x over previous
import functools

import jax
import jax.numpy as jnp
from jax import lax
from jax.experimental import pallas as pl
from jax.experimental.pallas import tpu as pltpu
from jax.experimental.pallas import tpu_sc as plsc

N = 10000
E = 320000
F_IN = 128
H = 128
C = 40

NC = 2
NS = 16
HH = H // NC
EPT = E // NS
K = 80
NCHT = EPT // K
NPAD = 10240
RPT = NPAD // NS
ZR = 128

BN = 2000

_MESH = dict(core_axis_name="c", subcore_axis_name="s", num_cores=NC,
             num_subcores=NS)


@functools.partial(
    pl.kernel,
    out_type=jax.ShapeDtypeStruct((NC, NPAD, HH), jnp.float32),
    mesh=plsc.VectorSubcoreMesh(**_MESH),
    scratch_types=[
        pltpu.VMEM((NCHT, K), jnp.int32),
        pltpu.VMEM((NCHT, K), jnp.int32),
        pltpu.VMEM((K,), jnp.int32),
        pltpu.VMEM((K, HH), jnp.float32),
        pltpu.VMEM((K, HH), jnp.float32),
        pltpu.VMEM((ZR, HH), jnp.float32),
        pltpu.VMEM_SHARED((NPAD, HH), jnp.float32),
        pltpu.SemaphoreType.DMA,
        pltpu.SemaphoreType.DMA,
    ],
    compiler_params=pltpu.CompilerParams(use_tc_tiling_on_sc=False),
)
def _agg_sc(src_hbm, dst_hbm, g_hbm, out_hbm, sidx, didx, dbuf, buf0, buf1,
            zv, acc, sem0, sem1):
    c = lax.axis_index("c")
    s = lax.axis_index("s")
    gc = g_hbm.at[c]

    pltpu.sync_copy(src_hbm.at[s], sidx)
    pltpu.sync_copy(dst_hbm.at[s], didx)

    @pl.loop(0, ZR)
    def _(r):
        for j in range(HH // 16):
            zv[r, pl.ds(j * 16, 16)] = jnp.zeros((16,), jnp.float32)

    for q in range(RPT // ZR):
        pltpu.sync_copy(zv, acc.at[pl.ds(s * RPT + q * ZR, ZR)])
    plsc.subcore_barrier()

    def scatter_chunk(i, buf):
        for j in range(K // 16):
            dbuf[pl.ds(j * 16, 16)] = didx[i, pl.ds(j * 16, 16)]
        pltpu.sync_copy(buf, acc.at[dbuf], add=True)

    @pl.loop(0, NCHT, step=2)
    def _(i):
        d0 = pltpu.async_copy(gc.at[sidx.at[i]], buf0, sem0)
        d1 = pltpu.async_copy(gc.at[sidx.at[i + 1]], buf1, sem1)
        d0.wait()
        scatter_chunk(i, buf0)
        d1.wait()
        scatter_chunk(i + 1, buf1)

    plsc.subcore_barrier()
    pltpu.sync_copy(acc.at[pl.ds(s * RPT, RPT)],
                    out_hbm.at[c, pl.ds(s * RPT, RPT)])


def _row_spec(w):
    return pl.BlockSpec((BN, w), lambda i: (i, 0))


def _split_spec():
    return pl.BlockSpec((NC, BN, HH), lambda i: (0, i, 0))


def _full_spec(a, b):
    return pl.BlockSpec((a, b), lambda i: (0, 0))


def _prep_body(dp, x, w, dinv_ref, g_ref):
    deg = dp[:, 0:1] + 1.0
    dinv = lax.rsqrt(deg)
    dinv_ref[...] = dinv
    g = dinv * jnp.dot(x[...], w[...], preferred_element_type=jnp.float32)
    g_ref[0] = g[:, :HH]
    g_ref[1] = g[:, HH:]


_prep_tc = pl.pallas_call(
    _prep_body,
    grid=(N // BN,),
    in_specs=[_row_spec(HH), _row_spec(F_IN), _full_spec(F_IN, H)],
    out_specs=[_row_spec(1), _split_spec()],
    out_shape=[jax.ShapeDtypeStruct((N, 1), jnp.float32),
               jax.ShapeDtypeStruct((NC, N, HH), jnp.float32)],
)


def _comb_body(a, g, dinv, b, w, m_in, m_ref, gn_ref):
    agg = jnp.concatenate([a[0], a[1]], axis=1)
    gg = jnp.concatenate([g[0], g[1]], axis=1)
    pre = dinv[...] * (agg + gg) + b[...]
    h = jnp.maximum(pre, 0.0)
    m_ref[...] = jnp.maximum(m_in[...], h)
    gn = dinv[...] * jnp.dot(h, w[...], preferred_element_type=jnp.float32)
    gn_ref[0] = gn[:, :HH]
    gn_ref[1] = gn[:, HH:]


_comb_tc = pl.pallas_call(
    _comb_body,
    grid=(N // BN,),
    in_specs=[_split_spec(), _split_spec(), _row_spec(1),
              _full_spec(1, H), _full_spec(H, H), _row_spec(H)],
    out_specs=[_row_spec(H), _split_spec()],
    out_shape=[jax.ShapeDtypeStruct((N, H), jnp.float32),
               jax.ShapeDtypeStruct((NC, N, HH), jnp.float32)],
)


def _final_body(a, g, dinv, b, m_in, wl, bl, out_ref):
    agg = jnp.concatenate([a[0], a[1]], axis=1)
    gg = jnp.concatenate([g[0], g[1]], axis=1)
    pre = dinv[...] * (agg + gg) + b[...]
    h = jnp.maximum(pre, 0.0)
    m = jnp.maximum(m_in[...], h)
    logits = jnp.dot(m, wl[...], preferred_element_type=jnp.float32) + bl[...]
    z = logits - jnp.max(logits, axis=1, keepdims=True)
    out_ref[...] = z - jnp.log(jnp.sum(jnp.exp(z), axis=1, keepdims=True))


_final_tc = pl.pallas_call(
    _final_body,
    grid=(N // BN,),
    in_specs=[_split_spec(), _split_spec(), _row_spec(1),
              _full_spec(1, H), _row_spec(H), _full_spec(H, C),
              _full_spec(1, C)],
    out_specs=_row_spec(C),
    out_shape=jax.ShapeDtypeStruct((N, C), jnp.float32),
)


def kernel(x, edge_index, W_in, b_in, W1, b1, W2, b2, W3, b3, Wl, bl):
    src3 = edge_index[0].astype(jnp.int32).reshape(NS, NCHT, K)
    dst3 = edge_index[1].astype(jnp.int32).reshape(NS, NCHT, K)

    ones_tab = jnp.ones((NC, N, HH), jnp.float32)
    deg_agg = _agg_sc(src3, dst3, ones_tab)
    dinv, g = _prep_tc(deg_agg[0], x, W_in)

    m = jnp.zeros((N, H), jnp.float32)
    for (W, b) in ((W1, b_in), (W2, b1), (W3, b2)):
        agg = _agg_sc(src3, dst3, g)
        m, g = _comb_tc(agg, g, dinv, b.reshape(1, H), W, m)

    agg = _agg_sc(src3, dst3, g)
    return _final_tc(agg, g, dinv, b3.reshape(1, H), m, Wl, bl.reshape(1, C))


# speedup vs baseline: 22.4967x; 1.8308x over previous
import functools

import jax
import jax.numpy as jnp
from jax import lax
from jax.experimental import pallas as pl
from jax.experimental.pallas import tpu as pltpu
from jax.experimental.pallas import tpu_sc as plsc

N = 10000
E = 320000
F_IN = 128
H = 128
C = 40

NC = 2
NS = 16
HH = H // NC
EPT = E // NS
K = 80
NCHT = EPT // K
NB = 5
NG = NCHT // NB
NPAD = 10240
RPT = NPAD // NS
ZR = 128

BN = 2000

_MESH = dict(core_axis_name="c", subcore_axis_name="s", num_cores=NC,
             num_subcores=NS)


@functools.partial(
    pl.kernel,
    out_type=jax.ShapeDtypeStruct((NC, NPAD, HH), jnp.float32),
    mesh=plsc.VectorSubcoreMesh(**_MESH),
    scratch_types=[
        pltpu.VMEM((NCHT, K), jnp.int32),
        [pltpu.VMEM((K,), jnp.int32) for _ in range(NB)],
        [pltpu.VMEM((K, HH), jnp.float32) for _ in range(NB)],
        pltpu.VMEM((ZR, HH), jnp.float32),
        pltpu.VMEM_SHARED((NPAD, HH), jnp.float32),
        [pltpu.SemaphoreType.DMA for _ in range(NB)],
        [pltpu.SemaphoreType.DMA for _ in range(NB)],
    ],
    compiler_params=pltpu.CompilerParams(use_tc_tiling_on_sc=False),
)
def _agg_sc(src_hbm, dst_hbm, g_hbm, out_hbm, sidx, dbufs, bufs, zv, acc,
            gsems, dsems):
    c = lax.axis_index("c")
    s = lax.axis_index("s")
    gc = g_hbm.at[c]

    pltpu.sync_copy(src_hbm.at[s], sidx)

    def start_chunk(j, b):
        pltpu.async_copy(gc.at[sidx.at[j]], bufs[b], gsems[b])
        pltpu.async_copy(dst_hbm.at[s, j], dbufs[b], dsems[b])

    def finish_chunk(j, b):
        pltpu.make_async_copy(gc.at[sidx.at[j]], bufs[b], gsems[b]).wait()
        pltpu.make_async_copy(dst_hbm.at[s, j], dbufs[b], dsems[b]).wait()
        pltpu.sync_copy(bufs[b], acc.at[dbufs[b]], add=True)

    @pl.loop(0, ZR)
    def _(r):
        for j in range(HH // 16):
            zv[r, pl.ds(j * 16, 16)] = jnp.zeros((16,), jnp.float32)

    for q in range(RPT // ZR):
        pltpu.sync_copy(zv, acc.at[pl.ds(s * RPT + q * ZR, ZR)])
    plsc.subcore_barrier()

    for b in range(NB - 1):
        start_chunk(b, b)

    @pl.loop(0, NG)
    def _(g):
        base = g * NB
        for b in range(NB):
            nxt = base + b + (NB - 1)
            pb = (b + NB - 1) % NB
            if b == 0:
                start_chunk(nxt, pb)
            else:
                @pl.when(g < NG - 1)
                def _():
                    start_chunk(nxt, pb)
            finish_chunk(base + b, b)

    plsc.subcore_barrier()
    pltpu.sync_copy(acc.at[pl.ds(s * RPT, RPT)],
                    out_hbm.at[c, pl.ds(s * RPT, RPT)])


def _row_spec(w):
    return pl.BlockSpec((BN, w), lambda i: (i, 0))


def _split_spec():
    return pl.BlockSpec((NC, BN, HH), lambda i: (0, i, 0))


def _full_spec(a, b):
    return pl.BlockSpec((a, b), lambda i: (0, 0))


def _prep_body(dp, x, w, dinv_ref, g_ref):
    deg = dp[:, 0:1] + 1.0
    dinv = lax.rsqrt(deg)
    dinv_ref[...] = dinv
    g = dinv * jnp.dot(x[...], w[...], preferred_element_type=jnp.float32)
    g_ref[0] = g[:, :HH]
    g_ref[1] = g[:, HH:]


_prep_tc = pl.pallas_call(
    _prep_body,
    grid=(N // BN,),
    in_specs=[_row_spec(HH), _row_spec(F_IN), _full_spec(F_IN, H)],
    out_specs=[_row_spec(1), _split_spec()],
    out_shape=[jax.ShapeDtypeStruct((N, 1), jnp.float32),
               jax.ShapeDtypeStruct((NC, N, HH), jnp.float32)],
)


def _comb_body(a, g, dinv, b, w, m_in, m_ref, gn_ref):
    agg = jnp.concatenate([a[0], a[1]], axis=1)
    gg = jnp.concatenate([g[0], g[1]], axis=1)
    pre = dinv[...] * (agg + gg) + b[...]
    h = jnp.maximum(pre, 0.0)
    m_ref[...] = jnp.maximum(m_in[...], h)
    gn = dinv[...] * jnp.dot(h, w[...], preferred_element_type=jnp.float32)
    gn_ref[0] = gn[:, :HH]
    gn_ref[1] = gn[:, HH:]


_comb_tc = pl.pallas_call(
    _comb_body,
    grid=(N // BN,),
    in_specs=[_split_spec(), _split_spec(), _row_spec(1),
              _full_spec(1, H), _full_spec(H, H), _row_spec(H)],
    out_specs=[_row_spec(H), _split_spec()],
    out_shape=[jax.ShapeDtypeStruct((N, H), jnp.float32),
               jax.ShapeDtypeStruct((NC, N, HH), jnp.float32)],
)


def _final_body(a, g, dinv, b, m_in, wl, bl, out_ref):
    agg = jnp.concatenate([a[0], a[1]], axis=1)
    gg = jnp.concatenate([g[0], g[1]], axis=1)
    pre = dinv[...] * (agg + gg) + b[...]
    h = jnp.maximum(pre, 0.0)
    m = jnp.maximum(m_in[...], h)
    logits = jnp.dot(m, wl[...], preferred_element_type=jnp.float32) + bl[...]
    z = logits - jnp.max(logits, axis=1, keepdims=True)
    out_ref[...] = z - jnp.log(jnp.sum(jnp.exp(z), axis=1, keepdims=True))


_final_tc = pl.pallas_call(
    _final_body,
    grid=(N // BN,),
    in_specs=[_split_spec(), _split_spec(), _row_spec(1),
              _full_spec(1, H), _row_spec(H), _full_spec(H, C),
              _full_spec(1, C)],
    out_specs=_row_spec(C),
    out_shape=jax.ShapeDtypeStruct((N, C), jnp.float32),
)


def kernel(x, edge_index, W_in, b_in, W1, b1, W2, b2, W3, b3, Wl, bl):
    src3 = edge_index[0].astype(jnp.int32).reshape(NS, NCHT, K)
    dst3 = edge_index[1].astype(jnp.int32).reshape(NS, NCHT, K)

    ones_tab = jnp.ones((NC, N, HH), jnp.float32)
    deg_agg = _agg_sc(src3, dst3, ones_tab)
    dinv, g = _prep_tc(deg_agg[0], x, W_in)

    m = jnp.zeros((N, H), jnp.float32)
    for (W, b) in ((W1, b_in), (W2, b1), (W3, b2)):
        agg = _agg_sc(src3, dst3, g)
        m, g = _comb_tc(agg, g, dinv, b.reshape(1, H), W, m)

    agg = _agg_sc(src3, dst3, g)
    return _final_tc(agg, g, dinv, b3.reshape(1, H), m, Wl, bl.reshape(1, C))


# speedup vs baseline: 23.6938x; 1.0532x over previous
import functools

import jax
import jax.numpy as jnp
from jax import lax
from jax.experimental import pallas as pl
from jax.experimental.pallas import tpu as pltpu
from jax.experimental.pallas import tpu_sc as plsc

N = 10000
E = 320000
F_IN = 128
H = 128
C = 40

NC = 2
NS = 16
HH = H // NC
K = 128
NB = 5
EPT = 20480
NCHT = EPT // K
NG = NCHT // NB
EPAD = NS * EPT
NPAD = 10240
RPT = NPAD // NS
ZR = 128

EPW = EPAD // (NC * NS)
NCHD = EPW // K

BN = 2000

_MESH = dict(core_axis_name="c", subcore_axis_name="s", num_cores=NC,
             num_subcores=NS)


@functools.partial(
    pl.kernel,
    out_type=jax.ShapeDtypeStruct((NC, NPAD, HH), jnp.float32),
    mesh=plsc.VectorSubcoreMesh(**_MESH),
    scratch_types=[
        pltpu.VMEM((NCHT, K), jnp.int32),
        [pltpu.VMEM((K,), jnp.int32) for _ in range(NB)],
        [pltpu.VMEM((K, HH), jnp.float32) for _ in range(NB)],
        pltpu.VMEM((ZR, HH), jnp.float32),
        pltpu.VMEM_SHARED((NPAD, HH), jnp.float32),
        [pltpu.SemaphoreType.DMA for _ in range(NB)],
        [pltpu.SemaphoreType.DMA for _ in range(NB)],
    ],
    compiler_params=pltpu.CompilerParams(use_tc_tiling_on_sc=False),
)
def _agg_sc(src_hbm, dst_hbm, g_hbm, out_hbm, sidx, dbufs, bufs, zv, acc,
            gsems, dsems):
    c = lax.axis_index("c")
    s = lax.axis_index("s")
    gc = g_hbm.at[c]

    pltpu.sync_copy(src_hbm.at[s], sidx)

    def start_chunk(j, b):
        pltpu.async_copy(gc.at[sidx.at[j]], bufs[b], gsems[b])
        pltpu.async_copy(dst_hbm.at[s, j], dbufs[b], dsems[b])

    def finish_chunk(j, b):
        pltpu.make_async_copy(gc.at[sidx.at[j]], bufs[b], gsems[b]).wait()
        pltpu.make_async_copy(dst_hbm.at[s, j], dbufs[b], dsems[b]).wait()
        pltpu.sync_copy(bufs[b], acc.at[dbufs[b]], add=True)

    @pl.loop(0, ZR)
    def _(r):
        for j in range(HH // 16):
            zv[r, pl.ds(j * 16, 16)] = jnp.zeros((16,), jnp.float32)

    for q in range(RPT // ZR):
        pltpu.sync_copy(zv, acc.at[pl.ds(s * RPT + q * ZR, ZR)])
    plsc.subcore_barrier()

    for b in range(NB - 1):
        start_chunk(b, b)

    @pl.loop(0, NG)
    def _(g):
        base = g * NB
        for b in range(NB):
            nxt = base + b + (NB - 1)
            pb = (b + NB - 1) % NB
            if b == 0:
                start_chunk(nxt, pb)
            else:
                @pl.when(g < NG - 1)
                def _():
                    start_chunk(nxt, pb)
            finish_chunk(base + b, b)

    plsc.subcore_barrier()
    pltpu.sync_copy(acc.at[pl.ds(s * RPT, RPT)],
                    out_hbm.at[c, pl.ds(s * RPT, RPT)])


@functools.partial(
    pl.kernel,
    out_type=jax.ShapeDtypeStruct((NC, NPAD, HH), jnp.float32),
    mesh=plsc.VectorSubcoreMesh(**_MESH),
    scratch_types=[
        [pltpu.VMEM((K,), jnp.int32) for _ in range(2)],
        pltpu.VMEM((K, HH), jnp.float32),
        pltpu.VMEM((ZR, HH), jnp.float32),
        pltpu.VMEM_SHARED((NPAD, HH), jnp.float32),
        [pltpu.SemaphoreType.DMA for _ in range(2)],
    ],
    compiler_params=pltpu.CompilerParams(use_tc_tiling_on_sc=False),
)
def _deg_sc(dst_hbm, ones_hbm, out_hbm, dbufs, ones_v, zv, acc, dsems):
    c = lax.axis_index("c")
    s = lax.axis_index("s")
    dmine = dst_hbm.at[s, c]

    pltpu.sync_copy(ones_hbm, ones_v)

    @pl.loop(0, ZR)
    def _(r):
        for j in range(HH // 16):
            zv[r, pl.ds(j * 16, 16)] = jnp.zeros((16,), jnp.float32)

    for q in range(RPT // ZR):
        pltpu.sync_copy(zv, acc.at[pl.ds(s * RPT + q * ZR, ZR)])
    plsc.subcore_barrier()

    pltpu.async_copy(dmine.at[0], dbufs[0], dsems[0])

    @pl.loop(0, NCHD)
    def _(i):
        for b in range(2):
            @pl.when(lax.rem(i, 2) == b)
            def _():
                @pl.when(i + 1 < NCHD)
                def _():
                    pltpu.async_copy(dmine.at[i + 1], dbufs[1 - b],
                                     dsems[1 - b])
                pltpu.make_async_copy(dmine.at[i], dbufs[b], dsems[b]).wait()
                pltpu.sync_copy(ones_v, acc.at[dbufs[b]], add=True)

    plsc.subcore_barrier()
    pltpu.sync_copy(acc.at[pl.ds(s * RPT, RPT)],
                    out_hbm.at[c, pl.ds(s * RPT, RPT)])


def _row_spec(w):
    return pl.BlockSpec((BN, w), lambda i: (i, 0))


def _split_spec():
    return pl.BlockSpec((NC, BN, HH), lambda i: (0, i, 0))


def _full_spec(a, b):
    return pl.BlockSpec((a, b), lambda i: (0, 0))


def _prep_body(dp0, dp1, x, w, dinv_ref, g_ref):
    deg = dp0[:, 0:1] + dp1[:, 0:1] + 1.0
    dinv = lax.rsqrt(deg)
    dinv_ref[...] = dinv
    g = dinv * jnp.dot(x[...], w[...], preferred_element_type=jnp.float32)
    g_ref[0] = g[:, :HH]
    g_ref[1] = g[:, HH:]


_prep_tc = pl.pallas_call(
    _prep_body,
    grid=(N // BN,),
    in_specs=[_row_spec(HH), _row_spec(HH), _row_spec(F_IN),
              _full_spec(F_IN, H)],
    out_specs=[_row_spec(1), _split_spec()],
    out_shape=[jax.ShapeDtypeStruct((N, 1), jnp.float32),
               jax.ShapeDtypeStruct((NC, N, HH), jnp.float32)],
)


def _comb_body(a, g, dinv, b, w, m_in, m_ref, gn_ref):
    agg = jnp.concatenate([a[0], a[1]], axis=1)
    gg = jnp.concatenate([g[0], g[1]], axis=1)
    pre = dinv[...] * (agg + gg) + b[...]
    h = jnp.maximum(pre, 0.0)
    m_ref[...] = jnp.maximum(m_in[...], h)
    gn = dinv[...] * jnp.dot(h, w[...], preferred_element_type=jnp.float32)
    gn_ref[0] = gn[:, :HH]
    gn_ref[1] = gn[:, HH:]


_comb_tc = pl.pallas_call(
    _comb_body,
    grid=(N // BN,),
    in_specs=[_split_spec(), _split_spec(), _row_spec(1),
              _full_spec(1, H), _full_spec(H, H), _row_spec(H)],
    out_specs=[_row_spec(H), _split_spec()],
    out_shape=[jax.ShapeDtypeStruct((N, H), jnp.float32),
               jax.ShapeDtypeStruct((NC, N, HH), jnp.float32)],
)


def _final_body(a, g, dinv, b, m_in, wl, bl, out_ref):
    agg = jnp.concatenate([a[0], a[1]], axis=1)
    gg = jnp.concatenate([g[0], g[1]], axis=1)
    pre = dinv[...] * (agg + gg) + b[...]
    h = jnp.maximum(pre, 0.0)
    m = jnp.maximum(m_in[...], h)
    logits = jnp.dot(m, wl[...], preferred_element_type=jnp.float32) + bl[...]
    z = logits - jnp.max(logits, axis=1, keepdims=True)
    out_ref[...] = z - jnp.log(jnp.sum(jnp.exp(z), axis=1, keepdims=True))


_final_tc = pl.pallas_call(
    _final_body,
    grid=(N // BN,),
    in_specs=[_split_spec(), _split_spec(), _row_spec(1),
              _full_spec(1, H), _row_spec(H), _full_spec(H, C),
              _full_spec(1, C)],
    out_specs=_row_spec(C),
    out_shape=jax.ShapeDtypeStruct((N, C), jnp.float32),
)


def kernel(x, edge_index, W_in, b_in, W1, b1, W2, b2, W3, b3, Wl, bl):
    src = edge_index[0].astype(jnp.int32)
    dst = edge_index[1].astype(jnp.int32)
    npd = EPAD - E
    pad_ids = jnp.arange(npd, dtype=jnp.int32)
    src = jnp.concatenate([src, pad_ids % N])
    dst = jnp.concatenate([dst, N + pad_ids % (NPAD - N)])
    src3 = src.reshape(NS, NCHT, K)
    dst3 = dst.reshape(NS, NCHT, K)
    dst4 = dst.reshape(NS, NC, NCHD, K)

    ones_row = jnp.ones((K, HH), jnp.float32)
    deg_parts = _deg_sc(dst4, ones_row)
    dinv, g = _prep_tc(deg_parts[0], deg_parts[1], x, W_in)

    m = jnp.zeros((N, H), jnp.float32)
    for (W, b) in ((W1, b_in), (W2, b1), (W3, b2)):
        agg = _agg_sc(src3, dst3, g)
        m, g = _comb_tc(agg, g, dinv, b.reshape(1, H), W, m)

    agg = _agg_sc(src3, dst3, g)
    return _final_tc(agg, g, dinv, b3.reshape(1, H), m, Wl, bl.reshape(1, C))


# speedup vs baseline: 23.7023x; 1.0004x over previous
import functools

import jax
import jax.numpy as jnp
from jax import lax
from jax.experimental import pallas as pl
from jax.experimental.pallas import tpu as pltpu
from jax.experimental.pallas import tpu_sc as plsc

N = 10000
E = 320000
F_IN = 128
H = 128
C = 40

NC = 2
NS = 16
HH = H // NC
K = 128
NB = 5
EPT = 20480
NCHT = EPT // K
NG = NCHT // NB
EPAD = NS * EPT
NPAD = 10240
RPT = NPAD // NS
ZR = 128

EPW = EPAD // (NC * NS)
NCHD = EPW // K

BN = 2000

_MESH = dict(core_axis_name="c", subcore_axis_name="s", num_cores=NC,
             num_subcores=NS)


@functools.partial(
    pl.kernel,
    out_type=jax.ShapeDtypeStruct((NC, NPAD, HH), jnp.float32),
    mesh=plsc.VectorSubcoreMesh(**_MESH),
    scratch_types=[
        pltpu.VMEM((NCHT, K), jnp.int32),
        [pltpu.VMEM((K,), jnp.int32) for _ in range(NB)],
        [pltpu.VMEM((K, HH), jnp.float32) for _ in range(NB)],
        pltpu.VMEM((ZR, HH), jnp.float32),
        pltpu.VMEM_SHARED((NPAD, HH), jnp.float32),
        [pltpu.SemaphoreType.DMA for _ in range(NB)],
        [pltpu.SemaphoreType.DMA for _ in range(NB)],
        [pltpu.SemaphoreType.DMA for _ in range(NB)],
    ],
    compiler_params=pltpu.CompilerParams(use_tc_tiling_on_sc=False),
)
def _agg_sc(src_hbm, dst_hbm, g_hbm, out_hbm, sidx, dbufs, bufs, zv, acc,
            gsems, dsems, ssems):
    c = lax.axis_index("c")
    s = lax.axis_index("s")
    gc = g_hbm.at[c]

    pltpu.sync_copy(src_hbm.at[s], sidx)

    def wait_scatter(b):
        pltpu.make_async_copy(bufs[b], acc.at[dbufs[b]], ssems[b]).wait()

    def start_chunk(j, b):
        pltpu.async_copy(gc.at[sidx.at[j]], bufs[b], gsems[b])
        pltpu.async_copy(dst_hbm.at[s, j], dbufs[b], dsems[b])

    def finish_chunk(j, b):
        pltpu.make_async_copy(gc.at[sidx.at[j]], bufs[b], gsems[b]).wait()
        pltpu.make_async_copy(dst_hbm.at[s, j], dbufs[b], dsems[b]).wait()
        pltpu.async_copy(bufs[b], acc.at[dbufs[b]], ssems[b], add=True)

    @pl.loop(0, ZR)
    def _(r):
        for j in range(HH // 16):
            zv[r, pl.ds(j * 16, 16)] = jnp.zeros((16,), jnp.float32)

    for q in range(RPT // ZR):
        pltpu.sync_copy(zv, acc.at[pl.ds(s * RPT + q * ZR, ZR)])
    plsc.subcore_barrier()

    for b in range(NB - 1):
        start_chunk(b, b)

    @pl.loop(0, NG)
    def _(g):
        base = g * NB
        for b in range(NB):
            nxt = base + b + (NB - 1)
            pb = (b + NB - 1) % NB
            if b == 0:
                @pl.when(g > 0)
                def _():
                    wait_scatter(pb)
                start_chunk(nxt, pb)
            else:
                @pl.when(g < NG - 1)
                def _():
                    wait_scatter(pb)
                    start_chunk(nxt, pb)
            finish_chunk(base + b, b)

    for b in range(NB):
        wait_scatter(b)
    plsc.subcore_barrier()
    pltpu.sync_copy(acc.at[pl.ds(s * RPT, RPT)],
                    out_hbm.at[c, pl.ds(s * RPT, RPT)])


@functools.partial(
    pl.kernel,
    out_type=jax.ShapeDtypeStruct((NC, NPAD, HH), jnp.float32),
    mesh=plsc.VectorSubcoreMesh(**_MESH),
    scratch_types=[
        [pltpu.VMEM((K,), jnp.int32) for _ in range(2)],
        pltpu.VMEM((K, HH), jnp.float32),
        pltpu.VMEM((ZR, HH), jnp.float32),
        pltpu.VMEM_SHARED((NPAD, HH), jnp.float32),
        [pltpu.SemaphoreType.DMA for _ in range(2)],
    ],
    compiler_params=pltpu.CompilerParams(use_tc_tiling_on_sc=False),
)
def _deg_sc(dst_hbm, ones_hbm, out_hbm, dbufs, ones_v, zv, acc, dsems):
    c = lax.axis_index("c")
    s = lax.axis_index("s")
    dmine = dst_hbm.at[s, c]

    pltpu.sync_copy(ones_hbm, ones_v)

    @pl.loop(0, ZR)
    def _(r):
        for j in range(HH // 16):
            zv[r, pl.ds(j * 16, 16)] = jnp.zeros((16,), jnp.float32)

    for q in range(RPT // ZR):
        pltpu.sync_copy(zv, acc.at[pl.ds(s * RPT + q * ZR, ZR)])
    plsc.subcore_barrier()

    pltpu.async_copy(dmine.at[0], dbufs[0], dsems[0])

    @pl.loop(0, NCHD)
    def _(i):
        for b in range(2):
            @pl.when(lax.rem(i, 2) == b)
            def _():
                @pl.when(i + 1 < NCHD)
                def _():
                    pltpu.async_copy(dmine.at[i + 1], dbufs[1 - b],
                                     dsems[1 - b])
                pltpu.make_async_copy(dmine.at[i], dbufs[b], dsems[b]).wait()
                pltpu.sync_copy(ones_v, acc.at[dbufs[b]], add=True)

    plsc.subcore_barrier()
    pltpu.sync_copy(acc.at[pl.ds(s * RPT, RPT)],
                    out_hbm.at[c, pl.ds(s * RPT, RPT)])


def _row_spec(w):
    return pl.BlockSpec((BN, w), lambda i: (i, 0))


def _split_spec():
    return pl.BlockSpec((NC, BN, HH), lambda i: (0, i, 0))


def _full_spec(a, b):
    return pl.BlockSpec((a, b), lambda i: (0, 0))


def _prep_body(dp0, dp1, x, w, dinv_ref, g_ref):
    deg = dp0[:, 0:1] + dp1[:, 0:1] + 1.0
    dinv = lax.rsqrt(deg)
    dinv_ref[...] = dinv
    g = dinv * jnp.dot(x[...], w[...], preferred_element_type=jnp.float32)
    g_ref[0] = g[:, :HH]
    g_ref[1] = g[:, HH:]


_prep_tc = pl.pallas_call(
    _prep_body,
    grid=(N // BN,),
    in_specs=[_row_spec(HH), _row_spec(HH), _row_spec(F_IN),
              _full_spec(F_IN, H)],
    out_specs=[_row_spec(1), _split_spec()],
    out_shape=[jax.ShapeDtypeStruct((N, 1), jnp.float32),
               jax.ShapeDtypeStruct((NC, N, HH), jnp.float32)],
)


def _comb_body(a, g, dinv, b, w, m_in, m_ref, gn_ref):
    agg = jnp.concatenate([a[0], a[1]], axis=1)
    gg = jnp.concatenate([g[0], g[1]], axis=1)
    pre = dinv[...] * (agg + gg) + b[...]
    h = jnp.maximum(pre, 0.0)
    m_ref[...] = jnp.maximum(m_in[...], h)
    gn = dinv[...] * jnp.dot(h, w[...], preferred_element_type=jnp.float32)
    gn_ref[0] = gn[:, :HH]
    gn_ref[1] = gn[:, HH:]


_comb_tc = pl.pallas_call(
    _comb_body,
    grid=(N // BN,),
    in_specs=[_split_spec(), _split_spec(), _row_spec(1),
              _full_spec(1, H), _full_spec(H, H), _row_spec(H)],
    out_specs=[_row_spec(H), _split_spec()],
    out_shape=[jax.ShapeDtypeStruct((N, H), jnp.float32),
               jax.ShapeDtypeStruct((NC, N, HH), jnp.float32)],
)


def _final_body(a, g, dinv, b, m_in, wl, bl, out_ref):
    agg = jnp.concatenate([a[0], a[1]], axis=1)
    gg = jnp.concatenate([g[0], g[1]], axis=1)
    pre = dinv[...] * (agg + gg) + b[...]
    h = jnp.maximum(pre, 0.0)
    m = jnp.maximum(m_in[...], h)
    logits = jnp.dot(m, wl[...], preferred_element_type=jnp.float32) + bl[...]
    z = logits - jnp.max(logits, axis=1, keepdims=True)
    out_ref[...] = z - jnp.log(jnp.sum(jnp.exp(z), axis=1, keepdims=True))


_final_tc = pl.pallas_call(
    _final_body,
    grid=(N // BN,),
    in_specs=[_split_spec(), _split_spec(), _row_spec(1),
              _full_spec(1, H), _row_spec(H), _full_spec(H, C),
              _full_spec(1, C)],
    out_specs=_row_spec(C),
    out_shape=jax.ShapeDtypeStruct((N, C), jnp.float32),
)


def kernel(x, edge_index, W_in, b_in, W1, b1, W2, b2, W3, b3, Wl, bl):
    src = edge_index[0].astype(jnp.int32)
    dst = edge_index[1].astype(jnp.int32)
    npd = EPAD - E
    pad_ids = jnp.arange(npd, dtype=jnp.int32)
    src = jnp.concatenate([src, pad_ids % N])
    dst = jnp.concatenate([dst, N + pad_ids % (NPAD - N)])
    src3 = src.reshape(NS, NCHT, K)
    dst3 = dst.reshape(NS, NCHT, K)
    dst4 = dst.reshape(NS, NC, NCHD, K)

    ones_row = jnp.ones((K, HH), jnp.float32)
    deg_parts = _deg_sc(dst4, ones_row)
    dinv, g = _prep_tc(deg_parts[0], deg_parts[1], x, W_in)

    m = jnp.zeros((N, H), jnp.float32)
    for (W, b) in ((W1, b_in), (W2, b1), (W3, b2)):
        agg = _agg_sc(src3, dst3, g)
        m, g = _comb_tc(agg, g, dinv, b.reshape(1, H), W, m)

    agg = _agg_sc(src3, dst3, g)
    return _final_tc(agg, g, dinv, b3.reshape(1, H), m, Wl, bl.reshape(1, C))
